# Initial kernel scaffold; baseline (speedup 1.0000x reference)
import jax
import jax.numpy as jnp
from jax.experimental import pallas as pl


def kernel(x, edge_index, edge_attr, batch, params):
    raise NotImplementedError("write your pallas kernel here")


# speedup vs baseline: 3.0679x; 3.0679x over previous
import functools

import jax
import jax.numpy as jnp
from jax import lax
from jax.experimental import pallas as pl
from jax.experimental.pallas import tpu as pltpu
from jax.experimental.pallas import tpu_sc as plsc

N = 10000
E = 320000
G = 64
IN_CH = 128
HID = 64
FC_W = 128
NCLS = 2

NC, NS = 2, 16
NW = NC * NS
K = 128
CPT = 80
EPT = K * CPT
EP = EPT * NW
NA = 10240
ZR = NA // NS


def _make_sc_agg(C):
    mesh = plsc.VectorSubcoreMesh(
        core_axis_name="c", subcore_axis_name="s", num_cores=NC,
        num_subcores=NS)

    @functools.partial(
        pl.kernel,
        out_type=jax.ShapeDtypeStruct((NC, NA, C), jnp.float32),
        mesh=mesh,
        scratch_types=[
            pltpu.VMEM((CPT, K), jnp.int32),
            pltpu.VMEM((K,), jnp.int32),
            pltpu.VMEM((K, C), jnp.float32),
            pltpu.VMEM_SHARED((NA, C), jnp.float32),
            pltpu.SemaphoreType.DMA,
        ],
        compiler_params=pltpu.CompilerParams(use_tc_tiling_on_sc=False),
    )
    def agg(table, zrows, srcp, dstf, out, src_v, dst_c, rows_v, acc, sem):
        cid = lax.axis_index("c")
        sid = lax.axis_index("s")
        wid = sid * NC + cid
        pltpu.sync_copy(zrows, acc.at[pl.ds(sid * ZR, ZR)])
        pltpu.sync_copy(srcp.at[pl.ds(wid * CPT, CPT)], src_v)
        plsc.subcore_barrier()

        def step(j, carry):
            pltpu.async_copy(table.at[src_v.at[j]], rows_v, sem).wait()
            pltpu.sync_copy(dstf.at[pl.ds(wid * EPT + j * K, K)], dst_c)
            pltpu.sync_copy(rows_v, acc.at[dst_c], add=True)
            return carry

        lax.fori_loop(0, CPT, step, 0)
        plsc.subcore_barrier()
        pltpu.sync_copy(acc.at[pl.ds(sid * ZR, ZR)],
                        out.at[cid, pl.ds(sid * ZR, ZR)])

    return agg


_SC_AGG_CACHE = {}


def _sc_agg(C):
    if C not in _SC_AGG_CACHE:
        _SC_AGG_CACHE[C] = _make_sc_agg(C)
    return _SC_AGG_CACHE[C]


def _l0_body(p_ref, x_ref, mwl, mwr, vwl, vwr, mb, vb, h_out, inv_out):
    p = (p_ref[0] + p_ref[1])[:N]
    inv = 1.0 / jnp.maximum(p[:, IN_CH:IN_CH + 1], 1.0)
    agg = p[:, :IN_CH] * inv
    x = x_ref[...]
    m = jnp.maximum(agg @ mwl[...] + mb[...] + x @ mwr[...], 0.0)
    v = jnp.maximum(agg @ vwl[...] + vb[...] + x @ vwr[...], 0.0)
    h_out[...] = jnp.concatenate([m, v], axis=1)
    inv_out[...] = inv


def _layer_body(p_ref, h_ref, inv_ref, mwl, mwr, vwl, vwr, mb, vb, h_out):
    p = (p_ref[0] + p_ref[1])[:N]
    inv = inv_ref[...]
    h = h_ref[...]
    m = jnp.maximum(p[:, :HID] * inv @ mwl[...] + mb[...]
                    + h[:, :HID] @ mwr[...], 0.0)
    v = jnp.maximum(p[:, HID:] * inv @ vwl[...] + vb[...]
                    + h[:, HID:] @ vwr[...], 0.0)
    h_out[...] = jnp.concatenate([m, v], axis=1)


def _head_body(h_ref, eps_ref, b_ref, f1w, f1b, f2w, f2b, out_ref):
    h = h_ref[...]
    z = h[:, :HID] + eps_ref[...] * jnp.exp(0.5 * h[:, HID:])
    gid = lax.broadcasted_iota(jnp.int32, (N, G), 1)
    oh = (b_ref[...] == gid).astype(jnp.float32)
    zsum = lax.dot_general(oh, z, (((0,), (0,)), ((), ())))
    cnt = jnp.sum(oh, axis=0)[:, None]
    zp = zsum / jnp.maximum(cnt, 1.0)
    a = jnp.maximum(zp @ f1w[...] + f1b[...], 0.0)
    logits = a @ f2w[...] + f2b[...]
    mx = jnp.max(logits, axis=1, keepdims=True)
    lse = mx + jnp.log(jnp.sum(jnp.exp(logits - mx), axis=1, keepdims=True))
    out_ref[...] = logits - lse


_EPS_CACHE = None


def _eps():
    global _EPS_CACHE
    if _EPS_CACHE is None:
        _EPS_CACHE = jax.random.normal(
            jax.random.key(42), (N, HID), jnp.float32)
    return _EPS_CACHE


def kernel(x, edge_index, edge_attr, batch, params):
    f32 = jnp.float32
    src = edge_index[0]
    dst = edge_index[1]
    srcp = jnp.concatenate(
        [src, jnp.zeros((EP - E,), jnp.int32)]).reshape(EP // K, K)
    dstp = jnp.concatenate([dst, jnp.full((EP - E,), N, jnp.int32)])
    z144 = jnp.zeros((ZR, IN_CH + 16), f32)
    z128 = jnp.zeros((ZR, 2 * HID), f32)

    table0 = jnp.concatenate(
        [x, jnp.ones((N, 1), f32), jnp.zeros((N, 15), f32)], axis=1)
    p0 = _sc_agg(IN_CH + 16)(table0, z144, srcp, dstp)

    b1 = lambda name: params[name].reshape(1, HID)
    h1, inv = pl.pallas_call(
        _l0_body,
        out_shape=(jax.ShapeDtypeStruct((N, 2 * HID), f32),
                   jax.ShapeDtypeStruct((N, 1), f32)),
    )(p0, x, params['mW_l0'], params['mW_r0'], params['vW_l0'],
      params['vW_r0'], b1('mb_l0'), b1('vb_l0'))

    h = h1
    for i in (1, 2):
        p = _sc_agg(2 * HID)(h, z128, srcp, dstp)
        h = pl.pallas_call(
            _layer_body,
            out_shape=jax.ShapeDtypeStruct((N, 2 * HID), f32),
        )(p, h, inv, params[f'mW_l{i}'], params[f'mW_r{i}'],
          params[f'vW_l{i}'], params[f'vW_r{i}'],
          b1(f'mb_l{i}'), b1(f'vb_l{i}'))

    logp = pl.pallas_call(
        _head_body,
        out_shape=jax.ShapeDtypeStruct((G, NCLS), f32),
    )(h, _eps(), batch.reshape(N, 1), params['fc1_W'],
      params['fc1_b'].reshape(1, FC_W), params['fc2_W'],
      params['fc2_b'].reshape(1, NCLS))

    return (logp, h[:, :HID], h[:, HID:])


# speedup vs baseline: 3.3560x; 1.0939x over previous
import functools

import jax
import jax.numpy as jnp
from jax import lax
from jax.experimental import pallas as pl
from jax.experimental.pallas import tpu as pltpu
from jax.experimental.pallas import tpu_sc as plsc

N = 10000
E = 320000
G = 64
IN_CH = 128
HID = 64
FC_W = 128
NCLS = 2

NC, NS = 2, 16
NW = NC * NS
K = 64
EPT = 10240
CPT = EPT // K
EP = EPT * NW
NA = 10016
ZR = NA // NS


def _make_sc_agg(C, NB):
    mesh = plsc.VectorSubcoreMesh(
        core_axis_name="c", subcore_axis_name="s", num_cores=NC,
        num_subcores=NS)

    @functools.partial(
        pl.kernel,
        out_type=jax.ShapeDtypeStruct((NC, NA, C), jnp.float32),
        mesh=mesh,
        scratch_types=(
            [pltpu.VMEM((CPT, K), jnp.int32)]
            + [pltpu.VMEM((K,), jnp.int32) for _ in range(NB)]
            + [pltpu.VMEM((K, C), jnp.float32) for _ in range(NB)]
            + [pltpu.VMEM_SHARED((NA, C), jnp.float32)]
            + [pltpu.SemaphoreType.DMA for _ in range(3 * NB)]
        ),
        compiler_params=pltpu.CompilerParams(use_tc_tiling_on_sc=False),
    )
    def agg(table, zrows, srcp, dstf, out, src_v, *rest):
        dst_c = rest[:NB]
        rows = rest[NB:2 * NB]
        acc = rest[2 * NB]
        sem_g = rest[2 * NB + 1:2 * NB + 1 + NB]
        sem_d = rest[2 * NB + 1 + NB:2 * NB + 1 + 2 * NB]
        sem_s = rest[2 * NB + 1 + 2 * NB:2 * NB + 1 + 3 * NB]
        cid = lax.axis_index("c")
        sid = lax.axis_index("s")
        wid = sid * NC + cid
        pltpu.sync_copy(zrows, acc.at[pl.ds(sid * ZR, ZR)])
        pltpu.sync_copy(srcp.at[pl.ds(wid * CPT, CPT)], src_v)
        plsc.subcore_barrier()

        def body(i, carry):
            jbase = i * NB
            started = []
            for b in range(NB):
                j = jbase + b
                g = pltpu.async_copy(table.at[src_v.at[j]], rows[b], sem_g[b])
                dd = pltpu.async_copy(
                    dstf.at[pl.ds(wid * EPT + j * K, K)], dst_c[b], sem_d[b])
                started.append((g, dd))
            scats = []
            for b in range(NB):
                g, dd = started[b]
                g.wait()
                dd.wait()
                scats.append(pltpu.async_copy(
                    rows[b], acc.at[dst_c[b]], sem_s[b], add=True))
            for s in scats:
                s.wait()
            return carry

        lax.fori_loop(0, CPT // NB, body, 0)
        plsc.subcore_barrier()
        pltpu.sync_copy(acc.at[pl.ds(sid * ZR, ZR)],
                        out.at[cid, pl.ds(sid * ZR, ZR)])

    return agg


_SC_AGG_CACHE = {}


def _sc_agg(C, NB):
    if (C, NB) not in _SC_AGG_CACHE:
        _SC_AGG_CACHE[(C, NB)] = _make_sc_agg(C, NB)
    return _SC_AGG_CACHE[(C, NB)]


def _l0_body(p_ref, x_ref, mwl, mwr, vwl, vwr, mb, vb, h_out, inv_out):
    p = (p_ref[0] + p_ref[1])[:N]
    inv = 1.0 / jnp.maximum(p[:, IN_CH:IN_CH + 1], 1.0)
    agg = p[:, :IN_CH] * inv
    x = x_ref[...]
    m = jnp.maximum(agg @ mwl[...] + mb[...] + x @ mwr[...], 0.0)
    v = jnp.maximum(agg @ vwl[...] + vb[...] + x @ vwr[...], 0.0)
    h_out[...] = jnp.concatenate([m, v], axis=1)
    inv_out[...] = inv


def _layer_body(p_ref, h_ref, inv_ref, mwl, mwr, vwl, vwr, mb, vb, h_out):
    p = (p_ref[0] + p_ref[1])[:N]
    inv = inv_ref[...]
    h = h_ref[...]
    m = jnp.maximum(p[:, :HID] * inv @ mwl[...] + mb[...]
                    + h[:, :HID] @ mwr[...], 0.0)
    v = jnp.maximum(p[:, HID:] * inv @ vwl[...] + vb[...]
                    + h[:, HID:] @ vwr[...], 0.0)
    h_out[...] = jnp.concatenate([m, v], axis=1)


def _head_body(h_ref, eps_ref, b_ref, f1w, f1b, f2w, f2b, out_ref):
    h = h_ref[...]
    z = h[:, :HID] + eps_ref[...] * jnp.exp(0.5 * h[:, HID:])
    gid = lax.broadcasted_iota(jnp.int32, (N, G), 1)
    oh = (b_ref[...] == gid).astype(jnp.float32)
    zsum = lax.dot_general(oh, z, (((0,), (0,)), ((), ())))
    cnt = jnp.sum(oh, axis=0)[:, None]
    zp = zsum / jnp.maximum(cnt, 1.0)
    a = jnp.maximum(zp @ f1w[...] + f1b[...], 0.0)
    logits = a @ f2w[...] + f2b[...]
    mx = jnp.max(logits, axis=1, keepdims=True)
    lse = mx + jnp.log(jnp.sum(jnp.exp(logits - mx), axis=1, keepdims=True))
    out_ref[...] = logits - lse


_EPS_CACHE = None


def _eps():
    global _EPS_CACHE
    if _EPS_CACHE is None:
        _EPS_CACHE = jax.random.normal(
            jax.random.key(42), (N, HID), jnp.float32)
    return _EPS_CACHE


def kernel(x, edge_index, edge_attr, batch, params):
    f32 = jnp.float32
    src = edge_index[0]
    dst = edge_index[1]
    srcp = jnp.concatenate(
        [src, jnp.zeros((EP - E,), jnp.int32)]).reshape(EP // K, K)
    dstp = jnp.concatenate([dst, jnp.full((EP - E,), N, jnp.int32)])
    z144 = jnp.zeros((ZR, IN_CH + 16), f32)
    z128 = jnp.zeros((ZR, 2 * HID), f32)

    table0 = jnp.concatenate(
        [x, jnp.ones((N, 1), f32), jnp.zeros((N, 15), f32)], axis=1)
    p0 = _sc_agg(IN_CH + 16, 3)(table0, z144, srcp, dstp)

    b1 = lambda name: params[name].reshape(1, HID)
    h1, inv = pl.pallas_call(
        _l0_body,
        out_shape=(jax.ShapeDtypeStruct((N, 2 * HID), f32),
                   jax.ShapeDtypeStruct((N, 1), f32)),
    )(p0, x, params['mW_l0'], params['mW_r0'], params['vW_l0'],
      params['vW_r0'], b1('mb_l0'), b1('vb_l0'))

    h = h1
    for i in (1, 2):
        p = _sc_agg(2 * HID, 4)(h, z128, srcp, dstp)
        h = pl.pallas_call(
            _layer_body,
            out_shape=jax.ShapeDtypeStruct((N, 2 * HID), f32),
        )(p, h, inv, params[f'mW_l{i}'], params[f'mW_r{i}'],
          params[f'vW_l{i}'], params[f'vW_r{i}'],
          b1(f'mb_l{i}'), b1(f'vb_l{i}'))

    logp = pl.pallas_call(
        _head_body,
        out_shape=jax.ShapeDtypeStruct((G, NCLS), f32),
    )(h, _eps(), batch.reshape(N, 1), params['fc1_W'],
      params['fc1_b'].reshape(1, FC_W), params['fc2_W'],
      params['fc2_b'].reshape(1, NCLS))

    return (logp, h[:, :HID], h[:, HID:])
